# baseline (speedup 1.0000x reference)
import functools

import jax
import jax.numpy as jnp
from jax import lax
from jax.experimental import pallas as pl
from jax.experimental.pallas import tpu as pltpu
from jax.experimental.pallas import tpu_sc as plsc

N = 10000
E = 320000
G = 64
DIN = 128
F = 64
OUTD = 10

NC, NS = 2, 16
NW = NC * NS
IB = 80
ROWS = E // IB
WROWS = ROWS // NW
OB = 5
NOUT = WROWS // OB
BE = OB * IB
NPW = N // NS

TB = 2000
NTB = N // TB

_MESH = plsc.VectorSubcoreMesh(
    core_axis_name="c", subcore_axis_name="s", num_cores=NC, num_subcores=NS)
_SC_PARAMS = pltpu.CompilerParams(
    use_tc_tiling_on_sc=False, needs_layout_passes=False)


def _lane_bcast(v, q):
  idx = jnp.full((16,), q, jnp.int32)
  return lax.gather(
      v, idx[:, None],
      lax.GatherDimensionNumbers(
          offset_dims=(), collapsed_slice_dims=(0,), start_index_map=(0,)),
      (1,), mode=lax.GatherScatterMode.PROMISE_IN_BOUNDS)


def _zero_shared(buf_v, acc_sh, s, width):
  zv = jnp.zeros((16,), jnp.float32)

  @pl.loop(0, BE)
  def _(r):
    for j in range(width // 16):
      buf_v[r, pl.ds(j * 16, 16)] = zv

  pltpu.sync_copy(buf_v.at[pl.ds(0, BE)], acc_sh.at[pl.ds(s * NPW, BE)])
  pltpu.sync_copy(buf_v.at[pl.ds(0, NPW - BE)],
                  acc_sh.at[pl.ds(s * NPW + BE, NPW - BE)])


def _writeback(acc_sh, out_hbm, c, s):
  pltpu.sync_copy(acc_sh.at[pl.ds(s * NPW, BE)],
                  out_hbm.at[c, pl.ds(s * NPW, BE)])
  pltpu.sync_copy(acc_sh.at[pl.ds(s * NPW + BE, NPW - BE)],
                  out_hbm.at[c, pl.ds(s * NPW + BE, NPW - BE)])


def _deg_body(dst_hbm, w_hbm, out_hbm, dst_a, dst_b, w_a, w_b,
              wrows_a, wrows_b, acc_sh, ssem_a, ssem_b):
  c = lax.axis_index("c")
  s = lax.axis_index("s")
  wid = c * NS + s
  base = wid * WROWS
  dst_v = (dst_a, dst_b)
  w_v = (w_a, w_b)
  wrows_v = (wrows_a, wrows_b)
  ssem = (ssem_a, ssem_b)
  _zero_shared(wrows_a, acc_sh, s, 16)
  plsc.subcore_barrier()

  def fetch(row0, b):
    pltpu.sync_copy(w_hbm.at[pl.ds(row0 * IB, BE)], w_v[b])
    for j in range(OB):
      pltpu.sync_copy(dst_hbm.at[pl.ds((row0 + j) * IB, IB)], dst_v[b].at[j])

  def build(b):
    @plsc.parallel_loop(0, BE // 16, unroll=2)
    def _(g, b=b):
      wv = jnp.abs(w_v[b][pl.ds(g * 16, 16)])
      for q in range(16):
        wrows_v[b][g * 16 + q, pl.ds(0, 16)] = jnp.full(
            (16,), wv[q], jnp.float32)

  def scatters(b):
    for j in range(OB):
      pltpu.async_copy(wrows_v[b].at[pl.ds(j * IB, IB)],
                       acc_sh.at[dst_v[b].at[j]], ssem[b], add=True)

  def sdrain(b):
    pltpu.make_async_copy(out_hbm.at[0, pl.ds(0, BE)],
                          wrows_v[b], ssem[b]).wait()

  fetch(base, 0)
  fetch(base + OB, 1)

  @pl.loop(0, NPAIR)
  def _(i):
    build(0)
    scatters(0)
    build(1)
    sdrain(0)
    fetch(base + (2 * i + 2) * OB, 0)
    scatters(1)
    sdrain(1)

    @pl.when(i < NPAIR - 1)
    def _():
      fetch(base + (2 * i + 3) * OB, 1)

  build(0)
  scatters(0)
  sdrain(0)

  plsc.subcore_barrier()
  _writeback(acc_sh, out_hbm, c, s)


def _deg_call(dst1, w1):
  return pl.kernel(
      _deg_body,
      out_type=jax.ShapeDtypeStruct((NC, N, 16), jnp.float32),
      mesh=_MESH,
      scratch_types=[
          pltpu.VMEM((OB, IB), jnp.int32),
          pltpu.VMEM((OB, IB), jnp.int32),
          pltpu.VMEM((BE,), jnp.float32),
          pltpu.VMEM((BE,), jnp.float32),
          pltpu.VMEM((BE, 16), jnp.float32),
          pltpu.VMEM((BE, 16), jnp.float32),
          pltpu.VMEM_SHARED((N, 16), jnp.float32),
          pltpu.SemaphoreType.DMA,
          pltpu.SemaphoreType.DMA,
      ],
      compiler_params=_SC_PARAMS,
  )(dst1, w1)


NPAIR = NOUT // 2


def _agg_body(xs_hbm, src_hbm, dst_hbm, w_hbm, out_hbm,
              src_a, src_b, dst_a, dst_b, w_a, w_b, rows_a, rows_b,
              acc_sh, gsem_a, gsem_b, ssem_a, ssem_b):
  c = lax.axis_index("c")
  s = lax.axis_index("s")
  wid = c * NS + s
  base = wid * WROWS
  src_v = (src_a, src_b)
  dst_v = (dst_a, dst_b)
  w_v = (w_a, w_b)
  rows_v = (rows_a, rows_b)
  gsem = (gsem_a, gsem_b)
  ssem = (ssem_a, ssem_b)
  _zero_shared(rows_a, acc_sh, s, F)
  plsc.subcore_barrier()

  def fetch(row0, b):
    pltpu.sync_copy(src_hbm.at[pl.ds(row0 * IB, BE)], src_v[b])
    pltpu.sync_copy(w_hbm.at[pl.ds(row0 * IB, BE)], w_v[b])
    for j in range(OB):
      pltpu.sync_copy(dst_hbm.at[pl.ds((row0 + j) * IB, IB)], dst_v[b].at[j])
    for j in range(OB):
      pltpu.async_copy(xs_hbm.at[src_v[b].at[pl.ds(j * IB, IB)]],
                       rows_v[b].at[pl.ds(j * IB, IB)], gsem[b])

  def gdrain(b):
    pltpu.make_async_copy(xs_hbm.at[pl.ds(0, BE)], rows_v[b], gsem[b]).wait()

  def scatters(b):
    for j in range(OB):
      pltpu.async_copy(rows_v[b].at[pl.ds(j * IB, IB)],
                       acc_sh.at[dst_v[b].at[j]], ssem[b], add=True)

  def sdrain(b):
    pltpu.make_async_copy(xs_hbm.at[pl.ds(0, BE)], rows_v[b], ssem[b]).wait()

  def multiply(b):
    @plsc.parallel_loop(0, BE // 16, unroll=2)
    def _(g, b=b):
      wv = jnp.abs(w_v[b][pl.ds(g * 16, 16)])
      for q in range(16):
        wvec = _lane_bcast(wv, q)
        r = g * 16 + q
        for col in range(F // 16):
          sl = (r, pl.ds(col * 16, 16))
          rows_v[b][sl] = rows_v[b][sl] * wvec

  fetch(base, 0)
  fetch(base + OB, 1)

  @pl.loop(0, NPAIR)
  def _(i):
    gdrain(0)
    multiply(0)
    scatters(0)
    gdrain(1)
    multiply(1)
    sdrain(0)
    fetch(base + (2 * i + 2) * OB, 0)
    scatters(1)
    sdrain(1)

    @pl.when(i < NPAIR - 1)
    def _():
      fetch(base + (2 * i + 3) * OB, 1)

  gdrain(0)
  multiply(0)
  scatters(0)
  sdrain(0)

  plsc.subcore_barrier()
  _writeback(acc_sh, out_hbm, c, s)


def _agg_call(xs, src1, dst1, w1):
  return pl.kernel(
      _agg_body,
      out_type=jax.ShapeDtypeStruct((NC, N, F), jnp.float32),
      mesh=_MESH,
      scratch_types=[
          pltpu.VMEM((BE,), jnp.int32),
          pltpu.VMEM((BE,), jnp.int32),
          pltpu.VMEM((OB, IB), jnp.int32),
          pltpu.VMEM((OB, IB), jnp.int32),
          pltpu.VMEM((BE,), jnp.float32),
          pltpu.VMEM((BE,), jnp.float32),
          pltpu.VMEM((BE, F), jnp.float32),
          pltpu.VMEM((BE, F), jnp.float32),
          pltpu.VMEM_SHARED((N, F), jnp.float32),
          pltpu.SemaphoreType.DMA,
          pltpu.SemaphoreType.DMA,
          pltpu.SemaphoreType.DMA,
          pltpu.SemaphoreType.DMA,
      ],
      compiler_params=_SC_PARAMS,
  )(xs, src1, dst1, w1)


def _prescale_body(dp_ref, x_ref, w_ref, dinvb_ref, xs_ref):
  deg = dp_ref[0, :, 0:1] + dp_ref[1, :, 0:1] + 1.0
  dinvb = jnp.broadcast_to(lax.rsqrt(deg), (TB, F))
  dinvb_ref[...] = dinvb
  xw = jnp.dot(x_ref[...], w_ref[...], preferred_element_type=jnp.float32)
  xs_ref[...] = xw * dinvb


def _prescale_call(deg_parts, x, W1):
  return pl.pallas_call(
      _prescale_body,
      grid=(NTB,),
      in_specs=[
          pl.BlockSpec((NC, TB, 16), lambda i: (0, i, 0)),
          pl.BlockSpec((TB, DIN), lambda i: (i, 0)),
          pl.BlockSpec((DIN, F), lambda i: (0, 0)),
      ],
      out_specs=[
          pl.BlockSpec((TB, F), lambda i: (i, 0)),
          pl.BlockSpec((TB, F), lambda i: (i, 0)),
      ],
      out_shape=[
          jax.ShapeDtypeStruct((N, F), jnp.float32),
          jax.ShapeDtypeStruct((N, F), jnp.float32),
      ],
  )(deg_parts, x, W1)


def _layer_body(acc_ref, xsp_ref, dinvb_ref, b_ref, w_ref, h_ref, xsn_ref):
  dinvb = dinvb_ref[...]
  tot = acc_ref[0] + acc_ref[1] + xsp_ref[...]
  h = jnp.maximum(dinvb * tot + b_ref[...], 0.0)
  h_ref[...] = h
  xsn_ref[...] = jnp.dot(h, w_ref[...],
                         preferred_element_type=jnp.float32) * dinvb


def _layer_call(acc_parts, xs_prev, dinvb, b_prev, W_next):
  return pl.pallas_call(
      _layer_body,
      grid=(NTB,),
      in_specs=[
          pl.BlockSpec((NC, TB, F), lambda i: (0, i, 0)),
          pl.BlockSpec((TB, F), lambda i: (i, 0)),
          pl.BlockSpec((TB, F), lambda i: (i, 0)),
          pl.BlockSpec((1, F), lambda i: (0, 0)),
          pl.BlockSpec((F, F), lambda i: (0, 0)),
      ],
      out_specs=[
          pl.BlockSpec((TB, F), lambda i: (i, 0)),
          pl.BlockSpec((TB, F), lambda i: (i, 0)),
      ],
      out_shape=[
          jax.ShapeDtypeStruct((N, F), jnp.float32),
          jax.ShapeDtypeStruct((N, F), jnp.float32),
      ],
  )(acc_parts, xs_prev, dinvb, b_prev, W_next)


def _final_body(acc_ref, xs3_ref, dinvb_ref, b3_ref, h1_ref, h2_ref,
                batch_ref, wl_ref, bl_ref, o_ref, accs, cnts):
  i = pl.program_id(0)

  @pl.when(i == 0)
  def _():
    accs[...] = jnp.zeros_like(accs)
    cnts[...] = jnp.zeros_like(cnts)

  dinvb = dinvb_ref[...]
  tot = acc_ref[0] + acc_ref[1] + xs3_ref[...]
  h3 = jnp.maximum(dinvb * tot + b3_ref[...], 0.0)
  xm = h1_ref[...] + h2_ref[...] + h3
  bvec = batch_ref[0, 0, :]
  oh = (bvec[None, :] == lax.broadcasted_iota(jnp.int32, (G, TB), 0)
        ).astype(jnp.float32)
  accs[...] += jnp.dot(oh, xm, preferred_element_type=jnp.float32)
  cnts[...] += jnp.broadcast_to(jnp.sum(oh, axis=1, keepdims=True), (G, F))

  @pl.when(i == NTB - 1)
  def _():
    pooled = accs[...] / (3.0 * jnp.maximum(cnts[...], 1.0))
    logits = jnp.dot(pooled, wl_ref[...],
                     preferred_element_type=jnp.float32) + bl_ref[...]
    m = jnp.max(logits, axis=1, keepdims=True)
    e = jnp.exp(logits - m)
    o_ref[...] = e / jnp.sum(e, axis=1, keepdims=True)


def _final_call(acc_parts, xs3, dinvb, b3, h1, h2, batch3, Wl, bl):
  return pl.pallas_call(
      _final_body,
      grid=(NTB,),
      in_specs=[
          pl.BlockSpec((NC, TB, F), lambda i: (0, i, 0)),
          pl.BlockSpec((TB, F), lambda i: (i, 0)),
          pl.BlockSpec((TB, F), lambda i: (i, 0)),
          pl.BlockSpec((1, F), lambda i: (0, 0)),
          pl.BlockSpec((TB, F), lambda i: (i, 0)),
          pl.BlockSpec((TB, F), lambda i: (i, 0)),
          pl.BlockSpec((1, 1, TB), lambda i: (i, 0, 0)),
          pl.BlockSpec((F, OUTD), lambda i: (0, 0)),
          pl.BlockSpec((1, OUTD), lambda i: (0, 0)),
      ],
      out_specs=pl.BlockSpec((G, OUTD), lambda i: (0, 0)),
      out_shape=jax.ShapeDtypeStruct((G, OUTD), jnp.float32),
      scratch_shapes=[
          pltpu.VMEM((G, F), jnp.float32),
          pltpu.VMEM((G, F), jnp.float32),
      ],
  )(acc_parts, xs3, dinvb, b3, h1, h2, batch3, Wl, bl)


def kernel(x, edge_index, edge_weight, batch, W1, b1, W2, b2, W3, b3, Wl, bl):
  src1 = edge_index[0]
  dst1 = edge_index[1]
  batch3 = batch.reshape(NTB, 1, TB)

  deg_parts = _deg_call(dst1, edge_weight)
  dinvb, xs1 = _prescale_call(deg_parts, x, W1)

  acc1 = _agg_call(xs1, src1, dst1, edge_weight)
  h1, xs2 = _layer_call(acc1, xs1, dinvb, b1.reshape(1, F), W2)
  acc2 = _agg_call(xs2, src1, dst1, edge_weight)
  h2, xs3 = _layer_call(acc2, xs2, dinvb, b2.reshape(1, F), W3)
  acc3 = _agg_call(xs3, src1, dst1, edge_weight)
  return _final_call(acc3, xs3, dinvb, b3.reshape(1, F), h1, h2,
                     batch3, Wl, bl.reshape(1, OUTD))


# speedup vs baseline: 1.3320x; 1.3320x over previous
import functools

import jax
import jax.numpy as jnp
from jax import lax
from jax.experimental import pallas as pl
from jax.experimental.pallas import tpu as pltpu
from jax.experimental.pallas import tpu_sc as plsc

N = 10000
E = 320000
G = 64
DIN = 128
F = 64
OUTD = 10

NC, NS = 2, 16
NW = NC * NS
IB = 80
ROWS = E // IB
WROWS = ROWS // NW
OB = 5
NOUT = WROWS // OB
BE = OB * IB
NPW = N // NS

TB = 2000
NTB = N // TB

_MESH = plsc.VectorSubcoreMesh(
    core_axis_name="c", subcore_axis_name="s", num_cores=NC, num_subcores=NS)
_SC_PARAMS = pltpu.CompilerParams(
    use_tc_tiling_on_sc=False, needs_layout_passes=False)


def _lane_bcast(v, q):
  idx = jnp.full((16,), q, jnp.int32)
  return lax.gather(
      v, idx[:, None],
      lax.GatherDimensionNumbers(
          offset_dims=(), collapsed_slice_dims=(0,), start_index_map=(0,)),
      (1,), mode=lax.GatherScatterMode.PROMISE_IN_BOUNDS)


def _zero_shared(buf_v, acc_sh, s, width):
  zv = jnp.zeros((16,), jnp.float32)

  @pl.loop(0, BE)
  def _(r):
    for j in range(width // 16):
      buf_v[r, pl.ds(j * 16, 16)] = zv

  pltpu.sync_copy(buf_v.at[pl.ds(0, BE)], acc_sh.at[pl.ds(s * NPW, BE)])
  pltpu.sync_copy(buf_v.at[pl.ds(0, NPW - BE)],
                  acc_sh.at[pl.ds(s * NPW + BE, NPW - BE)])


def _writeback(acc_sh, out_hbm, c, s):
  pltpu.sync_copy(acc_sh.at[pl.ds(s * NPW, BE)],
                  out_hbm.at[c, pl.ds(s * NPW, BE)])
  pltpu.sync_copy(acc_sh.at[pl.ds(s * NPW + BE, NPW - BE)],
                  out_hbm.at[c, pl.ds(s * NPW + BE, NPW - BE)])


def _deg_body(dst_hbm, w_hbm, out_hbm, dst_a, dst_b, w_a, w_b,
              wrows_a, wrows_b, acc_sh, ssem_a, ssem_b):
  c = lax.axis_index("c")
  s = lax.axis_index("s")
  wid = c * NS + s
  base = wid * WROWS
  dst_v = (dst_a, dst_b)
  w_v = (w_a, w_b)
  wrows_v = (wrows_a, wrows_b)
  ssem = (ssem_a, ssem_b)
  _zero_shared(wrows_a, acc_sh, s, 16)
  plsc.subcore_barrier()

  def fetch(row0, b):
    pltpu.sync_copy(w_hbm.at[pl.ds(row0 * IB, BE)], w_v[b])
    pltpu.sync_copy(dst_hbm.at[pl.ds(row0, OB)], dst_v[b])

  def build(b):
    @plsc.parallel_loop(0, BE // 16, unroll=2)
    def _(g, b=b):
      wv = jnp.abs(w_v[b][pl.ds(g * 16, 16)])
      for q in range(16):
        wrows_v[b][g * 16 + q, pl.ds(0, 16)] = jnp.full(
            (16,), wv[q], jnp.float32)

  def scatters(b):
    for j in range(OB):
      pltpu.async_copy(wrows_v[b].at[pl.ds(j * IB, IB)],
                       acc_sh.at[dst_v[b].at[j]], ssem[b], add=True)

  def sdrain(b):
    pltpu.make_async_copy(out_hbm.at[0, pl.ds(0, BE)],
                          wrows_v[b], ssem[b]).wait()

  fetch(base, 0)
  fetch(base + OB, 1)

  @pl.loop(0, NPAIR)
  def _(i):
    build(0)
    scatters(0)
    build(1)
    sdrain(0)
    fetch(base + (2 * i + 2) * OB, 0)
    scatters(1)
    sdrain(1)

    @pl.when(i < NPAIR - 1)
    def _():
      fetch(base + (2 * i + 3) * OB, 1)

  build(0)
  scatters(0)
  sdrain(0)

  plsc.subcore_barrier()
  _writeback(acc_sh, out_hbm, c, s)


def _deg_call(dst1, w1):
  return pl.kernel(
      _deg_body,
      out_type=jax.ShapeDtypeStruct((NC, N, 16), jnp.float32),
      mesh=_MESH,
      scratch_types=[
          pltpu.VMEM((OB, IB), jnp.int32),
          pltpu.VMEM((OB, IB), jnp.int32),
          pltpu.VMEM((BE,), jnp.float32),
          pltpu.VMEM((BE,), jnp.float32),
          pltpu.VMEM((BE, 16), jnp.float32),
          pltpu.VMEM((BE, 16), jnp.float32),
          pltpu.VMEM_SHARED((N, 16), jnp.float32),
          pltpu.SemaphoreType.DMA,
          pltpu.SemaphoreType.DMA,
      ],
      compiler_params=_SC_PARAMS,
  )(dst1, w1)


NPAIR = NOUT // 2


def _agg_body(xs_hbm, src_hbm, dst_hbm, w_hbm, out_hbm,
              src_a, src_b, dst_a, dst_b, w_a, w_b, rows_a, rows_b,
              acc_sh, gsem_a, gsem_b, ssem_a, ssem_b):
  c = lax.axis_index("c")
  s = lax.axis_index("s")
  wid = c * NS + s
  base = wid * WROWS
  src_v = (src_a, src_b)
  dst_v = (dst_a, dst_b)
  w_v = (w_a, w_b)
  rows_v = (rows_a, rows_b)
  gsem = (gsem_a, gsem_b)
  ssem = (ssem_a, ssem_b)
  _zero_shared(rows_a, acc_sh, s, F)
  plsc.subcore_barrier()

  def fetch(row0, b):
    pltpu.sync_copy(src_hbm.at[pl.ds(row0, OB)], src_v[b])
    pltpu.sync_copy(w_hbm.at[pl.ds(row0 * IB, BE)], w_v[b])
    pltpu.sync_copy(dst_hbm.at[pl.ds(row0, OB)], dst_v[b])
    for j in range(OB):
      pltpu.async_copy(xs_hbm.at[src_v[b].at[j]],
                       rows_v[b].at[pl.ds(j * IB, IB)], gsem[b])

  def gdrain(b):
    pltpu.make_async_copy(xs_hbm.at[pl.ds(0, BE)], rows_v[b], gsem[b]).wait()

  def scatters(b):
    for j in range(OB):
      pltpu.async_copy(rows_v[b].at[pl.ds(j * IB, IB)],
                       acc_sh.at[dst_v[b].at[j]], ssem[b], add=True)

  def sdrain(b):
    pltpu.make_async_copy(xs_hbm.at[pl.ds(0, BE)], rows_v[b], ssem[b]).wait()

  def multiply(b):
    @plsc.parallel_loop(0, BE // 16, unroll=2)
    def _(g, b=b):
      wv = jnp.abs(w_v[b][pl.ds(g * 16, 16)])
      for q in range(16):
        wvec = _lane_bcast(wv, q)
        r = g * 16 + q
        for col in range(F // 16):
          sl = (r, pl.ds(col * 16, 16))
          rows_v[b][sl] = rows_v[b][sl] * wvec

  fetch(base, 0)
  fetch(base + OB, 1)

  @pl.loop(0, NPAIR)
  def _(i):
    gdrain(0)
    multiply(0)
    scatters(0)
    gdrain(1)
    multiply(1)
    sdrain(0)
    fetch(base + (2 * i + 2) * OB, 0)
    scatters(1)
    sdrain(1)

    @pl.when(i < NPAIR - 1)
    def _():
      fetch(base + (2 * i + 3) * OB, 1)

  gdrain(0)
  multiply(0)
  scatters(0)
  sdrain(0)

  plsc.subcore_barrier()
  _writeback(acc_sh, out_hbm, c, s)


def _agg_call(xs, src1, dst1, w1):
  return pl.kernel(
      _agg_body,
      out_type=jax.ShapeDtypeStruct((NC, N, F), jnp.float32),
      mesh=_MESH,
      scratch_types=[
          pltpu.VMEM((OB, IB), jnp.int32),
          pltpu.VMEM((OB, IB), jnp.int32),
          pltpu.VMEM((OB, IB), jnp.int32),
          pltpu.VMEM((OB, IB), jnp.int32),
          pltpu.VMEM((BE,), jnp.float32),
          pltpu.VMEM((BE,), jnp.float32),
          pltpu.VMEM((BE, F), jnp.float32),
          pltpu.VMEM((BE, F), jnp.float32),
          pltpu.VMEM_SHARED((N, F), jnp.float32),
          pltpu.SemaphoreType.DMA,
          pltpu.SemaphoreType.DMA,
          pltpu.SemaphoreType.DMA,
          pltpu.SemaphoreType.DMA,
      ],
      compiler_params=_SC_PARAMS,
  )(xs, src1, dst1, w1)


def _prescale_body(dp_ref, x_ref, w_ref, dinvb_ref, xs_ref):
  deg = dp_ref[0, :, 0:1] + dp_ref[1, :, 0:1] + 1.0
  dinvb = jnp.broadcast_to(lax.rsqrt(deg), (TB, F))
  dinvb_ref[...] = dinvb
  xw = jnp.dot(x_ref[...], w_ref[...], preferred_element_type=jnp.float32)
  xs_ref[...] = xw * dinvb


def _prescale_call(deg_parts, x, W1):
  return pl.pallas_call(
      _prescale_body,
      grid=(NTB,),
      in_specs=[
          pl.BlockSpec((NC, TB, 16), lambda i: (0, i, 0)),
          pl.BlockSpec((TB, DIN), lambda i: (i, 0)),
          pl.BlockSpec((DIN, F), lambda i: (0, 0)),
      ],
      out_specs=[
          pl.BlockSpec((TB, F), lambda i: (i, 0)),
          pl.BlockSpec((TB, F), lambda i: (i, 0)),
      ],
      out_shape=[
          jax.ShapeDtypeStruct((N, F), jnp.float32),
          jax.ShapeDtypeStruct((N, F), jnp.float32),
      ],
  )(deg_parts, x, W1)


def _layer_body(acc_ref, xsp_ref, dinvb_ref, b_ref, w_ref, h_ref, xsn_ref):
  dinvb = dinvb_ref[...]
  tot = acc_ref[0] + acc_ref[1] + xsp_ref[...]
  h = jnp.maximum(dinvb * tot + b_ref[...], 0.0)
  h_ref[...] = h
  xsn_ref[...] = jnp.dot(h, w_ref[...],
                         preferred_element_type=jnp.float32) * dinvb


def _layer_call(acc_parts, xs_prev, dinvb, b_prev, W_next):
  return pl.pallas_call(
      _layer_body,
      grid=(NTB,),
      in_specs=[
          pl.BlockSpec((NC, TB, F), lambda i: (0, i, 0)),
          pl.BlockSpec((TB, F), lambda i: (i, 0)),
          pl.BlockSpec((TB, F), lambda i: (i, 0)),
          pl.BlockSpec((1, F), lambda i: (0, 0)),
          pl.BlockSpec((F, F), lambda i: (0, 0)),
      ],
      out_specs=[
          pl.BlockSpec((TB, F), lambda i: (i, 0)),
          pl.BlockSpec((TB, F), lambda i: (i, 0)),
      ],
      out_shape=[
          jax.ShapeDtypeStruct((N, F), jnp.float32),
          jax.ShapeDtypeStruct((N, F), jnp.float32),
      ],
  )(acc_parts, xs_prev, dinvb, b_prev, W_next)


def _final_body(acc_ref, xs3_ref, dinvb_ref, b3_ref, h1_ref, h2_ref,
                batch_ref, wl_ref, bl_ref, o_ref, accs, cnts):
  i = pl.program_id(0)

  @pl.when(i == 0)
  def _():
    accs[...] = jnp.zeros_like(accs)
    cnts[...] = jnp.zeros_like(cnts)

  dinvb = dinvb_ref[...]
  tot = acc_ref[0] + acc_ref[1] + xs3_ref[...]
  h3 = jnp.maximum(dinvb * tot + b3_ref[...], 0.0)
  xm = h1_ref[...] + h2_ref[...] + h3
  bvec = batch_ref[0, 0, :]
  oh = (bvec[None, :] == lax.broadcasted_iota(jnp.int32, (G, TB), 0)
        ).astype(jnp.float32)
  accs[...] += jnp.dot(oh, xm, preferred_element_type=jnp.float32)
  cnts[...] += jnp.broadcast_to(jnp.sum(oh, axis=1, keepdims=True), (G, F))

  @pl.when(i == NTB - 1)
  def _():
    pooled = accs[...] / (3.0 * jnp.maximum(cnts[...], 1.0))
    logits = jnp.dot(pooled, wl_ref[...],
                     preferred_element_type=jnp.float32) + bl_ref[...]
    m = jnp.max(logits, axis=1, keepdims=True)
    e = jnp.exp(logits - m)
    o_ref[...] = e / jnp.sum(e, axis=1, keepdims=True)


def _final_call(acc_parts, xs3, dinvb, b3, h1, h2, batch3, Wl, bl):
  return pl.pallas_call(
      _final_body,
      grid=(NTB,),
      in_specs=[
          pl.BlockSpec((NC, TB, F), lambda i: (0, i, 0)),
          pl.BlockSpec((TB, F), lambda i: (i, 0)),
          pl.BlockSpec((TB, F), lambda i: (i, 0)),
          pl.BlockSpec((1, F), lambda i: (0, 0)),
          pl.BlockSpec((TB, F), lambda i: (i, 0)),
          pl.BlockSpec((TB, F), lambda i: (i, 0)),
          pl.BlockSpec((1, 1, TB), lambda i: (i, 0, 0)),
          pl.BlockSpec((F, OUTD), lambda i: (0, 0)),
          pl.BlockSpec((1, OUTD), lambda i: (0, 0)),
      ],
      out_specs=pl.BlockSpec((G, OUTD), lambda i: (0, 0)),
      out_shape=jax.ShapeDtypeStruct((G, OUTD), jnp.float32),
      scratch_shapes=[
          pltpu.VMEM((G, F), jnp.float32),
          pltpu.VMEM((G, F), jnp.float32),
      ],
  )(acc_parts, xs3, dinvb, b3, h1, h2, batch3, Wl, bl)


def kernel(x, edge_index, edge_weight, batch, W1, b1, W2, b2, W3, b3, Wl, bl):
  src2 = edge_index[0].reshape(ROWS, IB)
  dst2 = edge_index[1].reshape(ROWS, IB)
  batch3 = batch.reshape(NTB, 1, TB)

  deg_parts = _deg_call(dst2, edge_weight)
  dinvb, xs1 = _prescale_call(deg_parts, x, W1)

  acc1 = _agg_call(xs1, src2, dst2, edge_weight)
  h1, xs2 = _layer_call(acc1, xs1, dinvb, b1.reshape(1, F), W2)
  acc2 = _agg_call(xs2, src2, dst2, edge_weight)
  h2, xs3 = _layer_call(acc2, xs2, dinvb, b2.reshape(1, F), W3)
  acc3 = _agg_call(xs3, src2, dst2, edge_weight)
  return _final_call(acc3, xs3, dinvb, b3.reshape(1, F), h1, h2,
                     batch3, Wl, bl.reshape(1, OUTD))
